# baseline (speedup 1.0000x reference)
import functools

import jax
import jax.numpy as jnp
from jax import lax
from jax.experimental import pallas as pl
from jax.experimental.pallas import tpu as pltpu
from jax.experimental.pallas import tpu_sc as plsc


def _gather_kernel(B, D, C):
    info = plsc.get_sparse_core_info()
    nw = info.num_cores * info.num_subcores
    b_per_w = B // nw
    n_chunks = b_per_w // C
    mesh = plsc.VectorSubcoreMesh(core_axis_name="c", subcore_axis_name="s")

    @functools.partial(
        pl.kernel,
        mesh=mesh,
        out_type=jax.ShapeDtypeStruct((B, D), jnp.float32),
        scratch_types=[
            pltpu.VMEM((C,), jnp.int32),
            pltpu.VMEM((C, D), jnp.float32),
            pltpu.SemaphoreType.DMA,
        ],
        compiler_params=pltpu.CompilerParams(use_tc_tiling_on_sc=False),
    )
    def k(idx_hbm, table_hbm, out_hbm, idx_v, rows_v, sem):
        wid = lax.axis_index("s") * info.num_cores + lax.axis_index("c")
        base = wid * b_per_w

        def body(g, carry):
            off = base + g * C
            pltpu.sync_copy(idx_hbm.at[pl.ds(off, C)], idx_v)
            pltpu.async_copy(table_hbm.at[idx_v], rows_v, sem).wait()
            pltpu.sync_copy(rows_v, out_hbm.at[pl.ds(off, C)])
            return carry

        lax.fori_loop(0, n_chunks, body, 0)

    return k


def kernel(x, emb_weight):
    B0, S = x.shape
    D = emb_weight.shape[1]
    B = B0 * S
    k = _gather_kernel(B, D, C=1024)
    out = k(x.reshape(B).astype(jnp.int32), emb_weight)
    return out.reshape(B0, S, D)


# speedup vs baseline: 1.0153x; 1.0153x over previous
import functools

import jax
import jax.numpy as jnp
from jax import lax
from jax.experimental import pallas as pl
from jax.experimental.pallas import tpu as pltpu
from jax.experimental.pallas import tpu_sc as plsc

_NBUF = 2


def _gather_kernel(B, D, C):
    info = plsc.get_sparse_core_info()
    nw = info.num_cores * info.num_subcores
    b_per_w = B // nw
    n_chunks = b_per_w // C
    n_groups = n_chunks // _NBUF
    mesh = plsc.VectorSubcoreMesh(core_axis_name="c", subcore_axis_name="s")

    @functools.partial(
        pl.kernel,
        mesh=mesh,
        out_type=jax.ShapeDtypeStruct((B, D), jnp.float32),
        scratch_types=[
            pltpu.VMEM((b_per_w,), jnp.int32),
            [pltpu.VMEM((C, D), jnp.float32) for _ in range(_NBUF)],
            [pltpu.SemaphoreType.DMA for _ in range(_NBUF)],
            [pltpu.SemaphoreType.DMA for _ in range(_NBUF)],
        ],
        compiler_params=pltpu.CompilerParams(use_tc_tiling_on_sc=False),
    )
    def k(idx_hbm, table_hbm, out_hbm, idx_v, rows, gsems, wsems):
        wid = lax.axis_index("s") * info.num_cores + lax.axis_index("c")
        base = wid * b_per_w
        pltpu.sync_copy(idx_hbm.at[pl.ds(base, b_per_w)], idx_v)

        def gather_start(g, b):
            pltpu.async_copy(
                table_hbm.at[idx_v.at[pl.ds(g * C, C)]], rows[b], gsems[b]
            )

        def gather_wait(g, b):
            pltpu.make_async_copy(
                table_hbm.at[idx_v.at[pl.ds(g * C, C)]], rows[b], gsems[b]
            ).wait()

        def write_start(g, b):
            pltpu.async_copy(rows[b], out_hbm.at[pl.ds(base + g * C, C)], wsems[b])

        def write_wait(g, b):
            pltpu.make_async_copy(
                rows[b], out_hbm.at[pl.ds(base + g * C, C)], wsems[b]
            ).wait()

        for b in range(_NBUF):
            gather_start(b, b)

        def body(gg, carry):
            for b in range(_NBUF):
                g = gg * _NBUF + b
                gather_wait(g, b)
                write_start(g, b)
                write_wait(g, b)
                gather_start(g + _NBUF, b)
            return carry

        lax.fori_loop(0, n_groups - 1, body, 0)

        for b in range(_NBUF):
            g = (n_groups - 1) * _NBUF + b
            gather_wait(g, b)
            write_start(g, b)
        for b in range(_NBUF):
            g = (n_groups - 1) * _NBUF + b
            write_wait(g, b)

    return k


def kernel(x, emb_weight):
    B0, S = x.shape
    D = emb_weight.shape[1]
    B = B0 * S
    k = _gather_kernel(B, D, C=640)
    out = k(x.reshape(B).astype(jnp.int32), emb_weight)
    return out.reshape(B0, S, D)
